# baseline (speedup 1.0000x reference)
import functools

import jax
import jax.numpy as jnp
from jax import lax
from jax.experimental import pallas as pl
from jax.experimental.pallas import tpu as pltpu
from jax.experimental.pallas import tpu_sc as plsc

N = 50000
E = 800000
KITER = 10
ALPHA = 0.1

NC = 2
NS = 16
LANES = 16

CW = 128
RPT = 420
EP = NS * RPT * CW
NROWS = EP // CW
RPTD = 196
NROWSD = 32 * RPTD
NP = 50048
NPT = NP // NS
UCH = 136
NCHU = NPT // UCH
ACC_ROWS = NP

_MESH = plsc.VectorSubcoreMesh(core_axis_name="c", subcore_axis_name="s")


@functools.partial(
    pl.kernel,
    out_type=jax.ShapeDtypeStruct((2, NP), jnp.float32),
    mesh=_MESH,
    compiler_params=pltpu.CompilerParams(use_tc_tiling_on_sc=False),
    scratch_types=[
        [pltpu.VMEM((CW,), jnp.int32)] * 4,
        pltpu.VMEM((CW,), jnp.float32),
        pltpu.VMEM((3136,), jnp.float32),
        pltpu.VMEM_SHARED((NP,), jnp.float32),
        [pltpu.SemaphoreType.DMA] * 4,
    ],
)
def _deg_kernel(dst_hbm, deg_hbm, didx, ones, zbuf, dacc, ssem):
    c = lax.axis_index("c")
    s = lax.axis_index("s")
    kbase = (c * NS + s) * RPTD

    for i in range(CW // LANES):
        ones[pl.ds(i * LANES, LANES)] = jnp.ones((LANES,), jnp.float32)
    for i in range(3136 // LANES):
        zbuf[pl.ds(i * LANES, LANES)] = jnp.zeros((LANES,), jnp.float32)
    pltpu.sync_copy(zbuf.at[pl.ds(0, 3128)], dacc.at[pl.ds(s * 3128, 3128)])
    plsc.subcore_barrier()

    @pl.loop(0, RPTD // 4)
    def _(j):
        for b in range(4):
            k = 4 * j + b

            @pl.when(j >= 1)
            def _():
                pltpu.make_async_copy(ones, dacc.at[didx[b]], ssem[b]).wait()

            pltpu.sync_copy(dst_hbm.at[kbase + k], didx[b])
            pltpu.async_copy(ones, dacc.at[didx[b]], ssem[b], add=True)

    for b in range(4):
        pltpu.make_async_copy(ones, dacc.at[didx[b]], ssem[b]).wait()
    plsc.subcore_barrier()
    pltpu.sync_copy(dacc.at[pl.ds(s * 3128, 3128)], zbuf.at[pl.ds(0, 3128)])
    pltpu.sync_copy(zbuf.at[pl.ds(0, 3128)],
                    deg_hbm.at[c, pl.ds(s * 3128, 3128)])


def _mlp_body(x_ref, dega_ref, degb_ref, w1_ref, b1_ref, w2_ref, b2_ref,
              ua_ref, ub_ref, qa_ref, qb_ref, d2w_ref, sq_ref):
    deg = dega_ref[...] + degb_ref[...] + 1.0
    dis = lax.rsqrt(deg)
    h = jnp.maximum(
        jnp.dot(x_ref[...], w1_ref[...].T, preferred_element_type=jnp.float32)
        + b1_ref[...][None, :], 0.0)
    z = (jnp.dot(h, w2_ref[...].T, preferred_element_type=jnp.float32)
         + b2_ref[...][None, :])
    u = dis * z
    q = (ALPHA / (1.0 - ALPHA)) * deg * u
    ua_ref[...] = u[:, :32]
    ub_ref[...] = u[:, 32:]
    qa_ref[...] = q[:, :32]
    qb_ref[...] = q[:, 32:]
    d2w_ref[...] = jnp.broadcast_to((1.0 - ALPHA) / deg, d2w_ref.shape)
    sq_ref[...] = deg * dis


@functools.partial(
    pl.kernel,
    out_type=jax.ShapeDtypeStruct((2 * NP, 32), jnp.float32),
    mesh=_MESH,
    compiler_params=pltpu.CompilerParams(use_tc_tiling_on_sc=False),
    scratch_types=[
        [pltpu.VMEM((2, CW), jnp.int32)] * 5,
        [pltpu.VMEM((CW, 32), jnp.float32)] * 5,
        pltpu.VMEM((UCH, 32), jnp.float32),
        pltpu.VMEM((UCH, 32), jnp.float32),
        pltpu.VMEM_SHARED((ACC_ROWS, 32), jnp.float32),
        [pltpu.SemaphoreType.DMA] * 5,
        [pltpu.SemaphoreType.DMA] * 5,
    ],
)
def _prop_kernel(sd_hbm, u0_hbm, q_hbm, d2w_hbm, u_hbm,
                 idx, rows, accbuf, dbuf, acc, gsem, ssem):
    c = lax.axis_index("c")
    s = lax.axis_index("s")
    cbase = c * NP
    kbase = s * RPT

    @pl.loop(0, NCHU)
    def _(j):
        r0 = s * NPT + j * UCH
        pltpu.sync_copy(q_hbm.at[pl.ds(cbase + r0, UCH)], acc.at[pl.ds(r0, UCH)])
    plsc.subcore_barrier()

    for it in range(KITER):
        table = u0_hbm if it == 0 else u_hbm

        def _ldidx(k, b):
            pltpu.sync_copy(sd_hbm.at[c, kbase + k], idx[b])

        def _gather(b):
            pltpu.async_copy(table.at[idx[b].at[0]], rows[b], gsem[b])

        for b in range(3):
            _ldidx(b, b)
            _gather(b)

        @pl.loop(0, RPT // 5)
        def _(j):
            for b in range(5):
                k = 5 * j + b
                b3 = (b + 3) % 5

                @pl.when(k >= 2)
                def _():
                    pltpu.make_async_copy(rows[b3], acc.at[idx[b3].at[1]],
                                          ssem[b3]).wait()

                @pl.when(k + 3 < RPT)
                def _():
                    _ldidx(k + 3, b3)
                    _gather(b3)

                pltpu.make_async_copy(table.at[idx[b].at[0]], rows[b],
                                      gsem[b]).wait()
                pltpu.async_copy(rows[b], acc.at[idx[b].at[1]], ssem[b],
                                 add=True)

        pltpu.make_async_copy(rows[3], acc.at[idx[3].at[1]], ssem[3]).wait()
        pltpu.make_async_copy(rows[4], acc.at[idx[4].at[1]], ssem[4]).wait()
        plsc.subcore_barrier()

        @pl.loop(0, NCHU)
        def _(j):
            r0 = s * NPT + j * UCH
            ra = cbase + r0
            pltpu.sync_copy(acc.at[pl.ds(r0, UCH)], accbuf)
            pltpu.sync_copy(q_hbm.at[pl.ds(ra, UCH)], acc.at[pl.ds(r0, UCH)])
            pltpu.sync_copy(d2w_hbm.at[pl.ds(r0, UCH)], dbuf)

            def _upd(r, _):
                for h in range(2):
                    sl = pl.ds(h * LANES, LANES)
                    accbuf[r, sl] = dbuf[r, sl] * accbuf[r, sl]
                return None
            lax.fori_loop(0, UCH, _upd, None)
            pltpu.sync_copy(accbuf, u_hbm.at[pl.ds(ra, UCH)])
        plsc.subcore_barrier()


def _final_body(ua_ref, ub_ref, sq_ref, o_ref):
    s = sq_ref[...]
    o_ref[:, :32] = ua_ref[...] * s
    o_ref[:, 32:] = ub_ref[...] * s


def kernel(x, edge_index, W1, b1, W2, b2):
    src = edge_index[0]
    dst = edge_index[1]
    loop = jnp.arange(NP, dtype=jnp.int32)
    pad = EP - E - NP
    padd = NROWSD * CW - E
    srcp = jnp.concatenate(
        [src, loop, jnp.zeros((pad,), jnp.int32)]).reshape(NROWS, CW)
    dstp = jnp.concatenate(
        [dst, loop, jnp.full((pad,), N, jnp.int32)]).reshape(NROWS, CW)
    sd = jnp.stack([
        jnp.stack([srcp, dstp], axis=1),
        jnp.stack([srcp + NP, dstp], axis=1),
    ])
    dstd = jnp.concatenate(
        [dst, jnp.full((padd,), N, jnp.int32)]).reshape(NROWSD, CW)

    degp = _deg_kernel(dstd)

    BN = 2000
    grid = (N // BN,)
    dega = degp[0, :N].reshape(N, 1)
    degb = degp[1, :N].reshape(N, 1)
    ua, ub, qa, qb, d2w, sq = pl.pallas_call(
        _mlp_body,
        grid=grid,
        in_specs=[
            pl.BlockSpec((BN, 128), lambda i: (i, 0)),
            pl.BlockSpec((BN, 1), lambda i: (i, 0)),
            pl.BlockSpec((BN, 1), lambda i: (i, 0)),
            pl.BlockSpec((128, 128), lambda i: (0, 0)),
            pl.BlockSpec((128,), lambda i: (0,)),
            pl.BlockSpec((64, 128), lambda i: (0, 0)),
            pl.BlockSpec((64,), lambda i: (0,)),
        ],
        out_specs=[
            pl.BlockSpec((BN, 32), lambda i: (i, 0)),
            pl.BlockSpec((BN, 32), lambda i: (i, 0)),
            pl.BlockSpec((BN, 32), lambda i: (i, 0)),
            pl.BlockSpec((BN, 32), lambda i: (i, 0)),
            pl.BlockSpec((BN, 32), lambda i: (i, 0)),
            pl.BlockSpec((BN, 1), lambda i: (i, 0)),
        ],
        out_shape=[
            jax.ShapeDtypeStruct((N, 32), jnp.float32),
            jax.ShapeDtypeStruct((N, 32), jnp.float32),
            jax.ShapeDtypeStruct((N, 32), jnp.float32),
            jax.ShapeDtypeStruct((N, 32), jnp.float32),
            jax.ShapeDtypeStruct((N, 32), jnp.float32),
            jax.ShapeDtypeStruct((N, 1), jnp.float32),
        ],
    )(x, dega, degb, W1, b1, W2, b2)

    zpad = jnp.zeros((NP - N, 32), jnp.float32)
    u0 = jnp.concatenate([ua, zpad, ub, zpad], axis=0)
    q = jnp.concatenate([qa, zpad, qb, zpad], axis=0)
    d2wp = jnp.concatenate([d2w, zpad], axis=0)

    uf = _prop_kernel(sd, u0, q, d2wp)
    ufa = uf[:N]
    ufb = uf[NP:NP + N]

    out = pl.pallas_call(
        _final_body,
        grid=grid,
        in_specs=[
            pl.BlockSpec((BN, 32), lambda i: (i, 0)),
            pl.BlockSpec((BN, 32), lambda i: (i, 0)),
            pl.BlockSpec((BN, 1), lambda i: (i, 0)),
        ],
        out_specs=pl.BlockSpec((BN, 64), lambda i: (i, 0)),
        out_shape=jax.ShapeDtypeStruct((N, 64), jnp.float32),
    )(ufa, ufb, sq)
    return out


# speedup vs baseline: 1.2177x; 1.2177x over previous
import functools

import jax
import jax.numpy as jnp
from jax import lax
from jax.experimental import pallas as pl
from jax.experimental.pallas import tpu as pltpu
from jax.experimental.pallas import tpu_sc as plsc

N = 50000
E = 800000
KITER = 10
ALPHA = 0.1

NC = 2
NS = 16
LANES = 16

CW = 128
RPT = 392
EP = NS * RPT * CW
NROWS = EP // CW
RPTD = 196
NROWSD = 32 * RPTD
NP = 50048
NPT = NP // NS
UCH = 136
NCHU = NPT // UCH
ACC_ROWS = NP

_MESH = plsc.VectorSubcoreMesh(core_axis_name="c", subcore_axis_name="s")


@functools.partial(
    pl.kernel,
    out_type=jax.ShapeDtypeStruct((2, NP), jnp.float32),
    mesh=_MESH,
    compiler_params=pltpu.CompilerParams(use_tc_tiling_on_sc=False),
    scratch_types=[
        [pltpu.VMEM((CW,), jnp.int32)] * 4,
        pltpu.VMEM((CW,), jnp.float32),
        pltpu.VMEM((3136,), jnp.float32),
        pltpu.VMEM_SHARED((NP,), jnp.float32),
        [pltpu.SemaphoreType.DMA] * 4,
    ],
)
def _deg_kernel(dst_hbm, deg_hbm, didx, ones, zbuf, dacc, ssem):
    c = lax.axis_index("c")
    s = lax.axis_index("s")
    kbase = (c * NS + s) * RPTD

    for i in range(CW // LANES):
        ones[pl.ds(i * LANES, LANES)] = jnp.ones((LANES,), jnp.float32)
    for i in range(3136 // LANES):
        zbuf[pl.ds(i * LANES, LANES)] = jnp.zeros((LANES,), jnp.float32)
    pltpu.sync_copy(zbuf.at[pl.ds(0, 3128)], dacc.at[pl.ds(s * 3128, 3128)])
    plsc.subcore_barrier()

    @pl.loop(0, RPTD // 4)
    def _(j):
        for b in range(4):
            k = 4 * j + b

            @pl.when(j >= 1)
            def _():
                pltpu.make_async_copy(ones, dacc.at[didx[b]], ssem[b]).wait()

            pltpu.sync_copy(dst_hbm.at[kbase + k], didx[b])
            pltpu.async_copy(ones, dacc.at[didx[b]], ssem[b], add=True)

    for b in range(4):
        pltpu.make_async_copy(ones, dacc.at[didx[b]], ssem[b]).wait()
    plsc.subcore_barrier()
    pltpu.sync_copy(dacc.at[pl.ds(s * 3128, 3128)], zbuf.at[pl.ds(0, 3128)])
    pltpu.sync_copy(zbuf.at[pl.ds(0, 3128)],
                    deg_hbm.at[c, pl.ds(s * 3128, 3128)])


def _mlp_body(x_ref, dega_ref, degb_ref, w1_ref, b1_ref, w2_ref, b2_ref,
              ua_ref, ub_ref, qa_ref, qb_ref, d2w_ref, sq_ref):
    deg = dega_ref[...] + degb_ref[...] + 1.0
    dis = lax.rsqrt(deg)
    h = jnp.maximum(
        jnp.dot(x_ref[...], w1_ref[...].T, preferred_element_type=jnp.float32)
        + b1_ref[...][None, :], 0.0)
    z = (jnp.dot(h, w2_ref[...].T, preferred_element_type=jnp.float32)
         + b2_ref[...][None, :])
    u = dis * z
    q = (ALPHA / (1.0 - ALPHA)) * deg * u
    ua_ref[...] = u[:, :32]
    ub_ref[...] = u[:, 32:]
    qa_ref[...] = q[:, :32]
    qb_ref[...] = q[:, 32:]
    d2w_ref[...] = jnp.broadcast_to((1.0 - ALPHA) / deg, d2w_ref.shape)
    sq_ref[...] = deg * dis


@functools.partial(
    pl.kernel,
    out_type=jax.ShapeDtypeStruct((2 * NP, 32), jnp.float32),
    mesh=_MESH,
    compiler_params=pltpu.CompilerParams(use_tc_tiling_on_sc=False),
    scratch_types=[
        [pltpu.VMEM((2, CW), jnp.int32)] * 4,
        [pltpu.VMEM((CW, 32), jnp.float32)] * 4,
        pltpu.VMEM((UCH, 32), jnp.float32),
        pltpu.VMEM((UCH, 32), jnp.float32),
        pltpu.VMEM((UCH, 32), jnp.float32),
        pltpu.VMEM_SHARED((ACC_ROWS, 32), jnp.float32),
        [pltpu.SemaphoreType.DMA] * 4,
        [pltpu.SemaphoreType.DMA] * 4,
    ],
)
def _prop_kernel(sd_hbm, u0_hbm, q_hbm, d2w_hbm, u_hbm,
                 idx, rows, accbuf, ubuf, dbuf, acc, gsem, ssem):
    c = lax.axis_index("c")
    s = lax.axis_index("s")
    cbase = c * NP
    kbase = s * RPT

    @pl.loop(0, NCHU)
    def _(j):
        r0 = s * NPT + j * UCH
        pltpu.sync_copy(q_hbm.at[pl.ds(cbase + r0, UCH)], acc.at[pl.ds(r0, UCH)])
    plsc.subcore_barrier()

    for it in range(KITER):
        table = u0_hbm if it == 0 else u_hbm

        def _ldidx(k, b):
            pltpu.sync_copy(sd_hbm.at[c, kbase + k], idx[b])

        def _gather(b):
            pltpu.async_copy(table.at[idx[b].at[0]], rows[b], gsem[b])

        for b in range(3):
            _ldidx(b, b)
        _gather(0)
        _gather(1)

        @pl.loop(0, RPT // 4)
        def _(j):
            for b in range(4):
                k = 4 * j + b
                b2 = (b + 2) % 4

                @pl.when(k >= 2)
                def _():
                    pltpu.make_async_copy(rows[b2], acc.at[idx[b2].at[1]],
                                          ssem[b2]).wait()

                @pl.when(k + 2 < RPT)
                def _():
                    _ldidx(k + 2, b2)
                    _gather(b2)

                pltpu.make_async_copy(table.at[idx[b].at[0]], rows[b],
                                      gsem[b]).wait()
                pltpu.async_copy(rows[b], acc.at[idx[b].at[1]], ssem[b],
                                 add=True)

        pltpu.make_async_copy(rows[2], acc.at[idx[2].at[1]], ssem[2]).wait()
        pltpu.make_async_copy(rows[3], acc.at[idx[3].at[1]], ssem[3]).wait()
        plsc.subcore_barrier()

        @pl.loop(0, NCHU)
        def _(j):
            r0 = s * NPT + j * UCH
            ra = cbase + r0
            pltpu.sync_copy(acc.at[pl.ds(r0, UCH)], accbuf)
            pltpu.sync_copy(q_hbm.at[pl.ds(ra, UCH)], acc.at[pl.ds(r0, UCH)])
            pltpu.sync_copy(table.at[pl.ds(ra, UCH)], ubuf)
            pltpu.sync_copy(d2w_hbm.at[pl.ds(r0, UCH)], dbuf)

            def _upd(r, _):
                for h in range(2):
                    sl = pl.ds(h * LANES, LANES)
                    accbuf[r, sl] = dbuf[r, sl] * (accbuf[r, sl] + ubuf[r, sl])
                return None
            lax.fori_loop(0, UCH, _upd, None)
            pltpu.sync_copy(accbuf, u_hbm.at[pl.ds(ra, UCH)])
        plsc.subcore_barrier()


def _final_body(ua_ref, ub_ref, sq_ref, o_ref):
    s = sq_ref[...]
    o_ref[:, :32] = ua_ref[...] * s
    o_ref[:, 32:] = ub_ref[...] * s


def kernel(x, edge_index, W1, b1, W2, b2):
    src = edge_index[0]
    dst = edge_index[1]
    pad = EP - E
    padd = NROWSD * CW - E
    srcp = jnp.concatenate(
        [src, jnp.zeros((pad,), jnp.int32)]).reshape(NROWS, CW)
    dstp = jnp.concatenate(
        [dst, jnp.full((pad,), N, jnp.int32)]).reshape(NROWS, CW)
    sd = jnp.stack([
        jnp.stack([srcp, dstp], axis=1),
        jnp.stack([srcp + NP, dstp], axis=1),
    ])
    dstd = jnp.concatenate(
        [dst, jnp.full((padd,), N, jnp.int32)]).reshape(NROWSD, CW)

    degp = _deg_kernel(dstd)

    BN = 2000
    grid = (N // BN,)
    dega = degp[0, :N].reshape(N, 1)
    degb = degp[1, :N].reshape(N, 1)
    ua, ub, qa, qb, d2w, sq = pl.pallas_call(
        _mlp_body,
        grid=grid,
        in_specs=[
            pl.BlockSpec((BN, 128), lambda i: (i, 0)),
            pl.BlockSpec((BN, 1), lambda i: (i, 0)),
            pl.BlockSpec((BN, 1), lambda i: (i, 0)),
            pl.BlockSpec((128, 128), lambda i: (0, 0)),
            pl.BlockSpec((128,), lambda i: (0,)),
            pl.BlockSpec((64, 128), lambda i: (0, 0)),
            pl.BlockSpec((64,), lambda i: (0,)),
        ],
        out_specs=[
            pl.BlockSpec((BN, 32), lambda i: (i, 0)),
            pl.BlockSpec((BN, 32), lambda i: (i, 0)),
            pl.BlockSpec((BN, 32), lambda i: (i, 0)),
            pl.BlockSpec((BN, 32), lambda i: (i, 0)),
            pl.BlockSpec((BN, 32), lambda i: (i, 0)),
            pl.BlockSpec((BN, 1), lambda i: (i, 0)),
        ],
        out_shape=[
            jax.ShapeDtypeStruct((N, 32), jnp.float32),
            jax.ShapeDtypeStruct((N, 32), jnp.float32),
            jax.ShapeDtypeStruct((N, 32), jnp.float32),
            jax.ShapeDtypeStruct((N, 32), jnp.float32),
            jax.ShapeDtypeStruct((N, 32), jnp.float32),
            jax.ShapeDtypeStruct((N, 1), jnp.float32),
        ],
    )(x, dega, degb, W1, b1, W2, b2)

    zpad = jnp.zeros((NP - N, 32), jnp.float32)
    u0 = jnp.concatenate([ua, zpad, ub, zpad], axis=0)
    q = jnp.concatenate([qa, zpad, qb, zpad], axis=0)
    d2wp = jnp.concatenate([d2w, zpad], axis=0)

    uf = _prop_kernel(sd, u0, q, d2wp)
    ufa = uf[:N]
    ufb = uf[NP:NP + N]

    out = pl.pallas_call(
        _final_body,
        grid=grid,
        in_specs=[
            pl.BlockSpec((BN, 32), lambda i: (i, 0)),
            pl.BlockSpec((BN, 32), lambda i: (i, 0)),
            pl.BlockSpec((BN, 1), lambda i: (i, 0)),
        ],
        out_specs=pl.BlockSpec((BN, 64), lambda i: (i, 0)),
        out_shape=jax.ShapeDtypeStruct((N, 64), jnp.float32),
    )(ufa, ufb, sq)
    return out


# speedup vs baseline: 1.2773x; 1.0490x over previous
import functools

import jax
import jax.numpy as jnp
from jax import lax
from jax.experimental import pallas as pl
from jax.experimental.pallas import tpu as pltpu
from jax.experimental.pallas import tpu_sc as plsc

N = 50000
E = 800000
KITER = 10
ALPHA = 0.1

NC = 2
NS = 16
LANES = 16

CW = 128
RPT = 196
EP = NS * RPT * 2 * CW
NROWS = EP // CW
RPTD = 196
NROWSD = 32 * RPTD
NP = 50048
NPT = NP // NS
UCH = 136
NCHU = NPT // UCH
ACC_ROWS = NP

_MESH = plsc.VectorSubcoreMesh(core_axis_name="c", subcore_axis_name="s")


@functools.partial(
    pl.kernel,
    out_type=jax.ShapeDtypeStruct((2, NP), jnp.float32),
    mesh=_MESH,
    compiler_params=pltpu.CompilerParams(use_tc_tiling_on_sc=False),
    scratch_types=[
        [pltpu.VMEM((CW,), jnp.int32)] * 4,
        pltpu.VMEM((CW,), jnp.float32),
        pltpu.VMEM((3136,), jnp.float32),
        pltpu.VMEM_SHARED((NP,), jnp.float32),
        [pltpu.SemaphoreType.DMA] * 4,
    ],
)
def _deg_kernel(dst_hbm, deg_hbm, didx, ones, zbuf, dacc, ssem):
    c = lax.axis_index("c")
    s = lax.axis_index("s")
    kbase = (c * NS + s) * RPTD

    for i in range(CW // LANES):
        ones[pl.ds(i * LANES, LANES)] = jnp.ones((LANES,), jnp.float32)
    for i in range(3136 // LANES):
        zbuf[pl.ds(i * LANES, LANES)] = jnp.zeros((LANES,), jnp.float32)
    pltpu.sync_copy(zbuf.at[pl.ds(0, 3128)], dacc.at[pl.ds(s * 3128, 3128)])
    plsc.subcore_barrier()

    @pl.loop(0, RPTD // 4)
    def _(j):
        for b in range(4):
            k = 4 * j + b

            @pl.when(j >= 1)
            def _():
                pltpu.make_async_copy(ones, dacc.at[didx[b]], ssem[b]).wait()

            pltpu.sync_copy(dst_hbm.at[kbase + k], didx[b])
            pltpu.async_copy(ones, dacc.at[didx[b]], ssem[b], add=True)

    for b in range(4):
        pltpu.make_async_copy(ones, dacc.at[didx[b]], ssem[b]).wait()
    plsc.subcore_barrier()
    pltpu.sync_copy(dacc.at[pl.ds(s * 3128, 3128)], zbuf.at[pl.ds(0, 3128)])
    pltpu.sync_copy(zbuf.at[pl.ds(0, 3128)],
                    deg_hbm.at[c, pl.ds(s * 3128, 3128)])


def _mlp_body(x_ref, dega_ref, degb_ref, w1_ref, b1_ref, w2_ref, b2_ref,
              ua_ref, ub_ref, qa_ref, qb_ref, d2w_ref, sq_ref):
    deg = dega_ref[...] + degb_ref[...] + 1.0
    dis = lax.rsqrt(deg)
    h = jnp.maximum(
        jnp.dot(x_ref[...], w1_ref[...].T, preferred_element_type=jnp.float32)
        + b1_ref[...][None, :], 0.0)
    z = (jnp.dot(h, w2_ref[...].T, preferred_element_type=jnp.float32)
         + b2_ref[...][None, :])
    u = dis * z
    q = (ALPHA / (1.0 - ALPHA)) * deg * u
    ua_ref[...] = u[:, :32]
    ub_ref[...] = u[:, 32:]
    qa_ref[...] = q[:, :32]
    qb_ref[...] = q[:, 32:]
    d2w_ref[...] = jnp.broadcast_to((1.0 - ALPHA) / deg, d2w_ref.shape)
    sq_ref[...] = deg * dis


@functools.partial(
    pl.kernel,
    out_type=jax.ShapeDtypeStruct((2 * NP, 32), jnp.float32),
    mesh=_MESH,
    compiler_params=pltpu.CompilerParams(use_tc_tiling_on_sc=False),
    scratch_types=[
        [pltpu.VMEM((2, 2 * CW), jnp.int32)] * 2,
        [pltpu.VMEM((2 * CW, 32), jnp.float32)] * 2,
        pltpu.VMEM((UCH, 32), jnp.float32),
        pltpu.VMEM((UCH, 32), jnp.float32),
        pltpu.VMEM((UCH, 32), jnp.float32),
        pltpu.VMEM_SHARED((ACC_ROWS, 32), jnp.float32),
        [pltpu.SemaphoreType.DMA] * 2,
        [pltpu.SemaphoreType.DMA] * 2,
    ],
)
def _prop_kernel(sd_hbm, u0_hbm, q_hbm, d2w_hbm, u_hbm,
                 idx, rows, accbuf, ubuf, dbuf, acc, gsem, ssem):
    c = lax.axis_index("c")
    s = lax.axis_index("s")
    cbase = c * NP
    kbase = s * RPT

    @pl.loop(0, NCHU)
    def _(j):
        r0 = s * NPT + j * UCH
        pltpu.sync_copy(q_hbm.at[pl.ds(cbase + r0, UCH)], acc.at[pl.ds(r0, UCH)])
    plsc.subcore_barrier()

    for it in range(KITER):
        table = u0_hbm if it == 0 else u_hbm

        def _ldidx(k, b):
            pltpu.sync_copy(sd_hbm.at[c, kbase + k], idx[b])

        def _gather(b):
            pltpu.async_copy(table.at[idx[b].at[0]], rows[b], gsem[b])

        _ldidx(0, 0)
        _gather(0)

        @pl.loop(0, RPT // 2)
        def _(j):
            for b in range(2):
                k = 2 * j + b
                b1 = (b + 1) % 2

                @pl.when(k >= 1)
                def _():
                    pltpu.make_async_copy(rows[b1], acc.at[idx[b1].at[1]],
                                          ssem[b1]).wait()

                @pl.when(k + 1 < RPT)
                def _():
                    _ldidx(k + 1, b1)
                    _gather(b1)

                pltpu.make_async_copy(table.at[idx[b].at[0]], rows[b],
                                      gsem[b]).wait()
                pltpu.async_copy(rows[b], acc.at[idx[b].at[1]], ssem[b],
                                 add=True)

        pltpu.make_async_copy(rows[(RPT - 1) % 2],
                              acc.at[idx[(RPT - 1) % 2].at[1]],
                              ssem[(RPT - 1) % 2]).wait()
        plsc.subcore_barrier()

        @pl.loop(0, NCHU)
        def _(j):
            r0 = s * NPT + j * UCH
            ra = cbase + r0
            pltpu.sync_copy(acc.at[pl.ds(r0, UCH)], accbuf)
            pltpu.sync_copy(q_hbm.at[pl.ds(ra, UCH)], acc.at[pl.ds(r0, UCH)])
            pltpu.sync_copy(table.at[pl.ds(ra, UCH)], ubuf)
            pltpu.sync_copy(d2w_hbm.at[pl.ds(r0, UCH)], dbuf)

            def _upd(r, _):
                for h in range(2):
                    sl = pl.ds(h * LANES, LANES)
                    accbuf[r, sl] = dbuf[r, sl] * (accbuf[r, sl] + ubuf[r, sl])
                return None
            lax.fori_loop(0, UCH, _upd, None)
            pltpu.sync_copy(accbuf, u_hbm.at[pl.ds(ra, UCH)])
        plsc.subcore_barrier()


def _final_body(ua_ref, ub_ref, sq_ref, o_ref):
    s = sq_ref[...]
    o_ref[:, :32] = ua_ref[...] * s
    o_ref[:, 32:] = ub_ref[...] * s


def kernel(x, edge_index, W1, b1, W2, b2):
    src = edge_index[0]
    dst = edge_index[1]
    pad = EP - E
    padd = NROWSD * CW - E
    srcp = jnp.concatenate(
        [src, jnp.zeros((pad,), jnp.int32)]).reshape(NROWS, CW)
    dstp = jnp.concatenate(
        [dst, jnp.full((pad,), N, jnp.int32)]).reshape(NROWS, CW)
    srcp2 = srcp.reshape(NROWS // 2, 2 * CW)
    dstp2 = dstp.reshape(NROWS // 2, 2 * CW)
    sd = jnp.stack([
        jnp.stack([srcp2, dstp2], axis=1),
        jnp.stack([srcp2 + NP, dstp2], axis=1),
    ])
    dstd = jnp.concatenate(
        [dst, jnp.full((padd,), N, jnp.int32)]).reshape(NROWSD, CW)

    degp = _deg_kernel(dstd)

    BN = 2000
    grid = (N // BN,)
    dega = degp[0, :N].reshape(N, 1)
    degb = degp[1, :N].reshape(N, 1)
    ua, ub, qa, qb, d2w, sq = pl.pallas_call(
        _mlp_body,
        grid=grid,
        in_specs=[
            pl.BlockSpec((BN, 128), lambda i: (i, 0)),
            pl.BlockSpec((BN, 1), lambda i: (i, 0)),
            pl.BlockSpec((BN, 1), lambda i: (i, 0)),
            pl.BlockSpec((128, 128), lambda i: (0, 0)),
            pl.BlockSpec((128,), lambda i: (0,)),
            pl.BlockSpec((64, 128), lambda i: (0, 0)),
            pl.BlockSpec((64,), lambda i: (0,)),
        ],
        out_specs=[
            pl.BlockSpec((BN, 32), lambda i: (i, 0)),
            pl.BlockSpec((BN, 32), lambda i: (i, 0)),
            pl.BlockSpec((BN, 32), lambda i: (i, 0)),
            pl.BlockSpec((BN, 32), lambda i: (i, 0)),
            pl.BlockSpec((BN, 32), lambda i: (i, 0)),
            pl.BlockSpec((BN, 1), lambda i: (i, 0)),
        ],
        out_shape=[
            jax.ShapeDtypeStruct((N, 32), jnp.float32),
            jax.ShapeDtypeStruct((N, 32), jnp.float32),
            jax.ShapeDtypeStruct((N, 32), jnp.float32),
            jax.ShapeDtypeStruct((N, 32), jnp.float32),
            jax.ShapeDtypeStruct((N, 32), jnp.float32),
            jax.ShapeDtypeStruct((N, 1), jnp.float32),
        ],
    )(x, dega, degb, W1, b1, W2, b2)

    zpad = jnp.zeros((NP - N, 32), jnp.float32)
    u0 = jnp.concatenate([ua, zpad, ub, zpad], axis=0)
    q = jnp.concatenate([qa, zpad, qb, zpad], axis=0)
    d2wp = jnp.concatenate([d2w, zpad], axis=0)

    uf = _prop_kernel(sd, u0, q, d2wp)
    ufa = uf[:N]
    ufb = uf[NP:NP + N]

    out = pl.pallas_call(
        _final_body,
        grid=grid,
        in_specs=[
            pl.BlockSpec((BN, 32), lambda i: (i, 0)),
            pl.BlockSpec((BN, 32), lambda i: (i, 0)),
            pl.BlockSpec((BN, 1), lambda i: (i, 0)),
        ],
        out_specs=pl.BlockSpec((BN, 64), lambda i: (i, 0)),
        out_shape=jax.ShapeDtypeStruct((N, 64), jnp.float32),
    )(ufa, ufb, sq)
    return out
